# baseline (speedup 1.0000x reference)
import dataclasses
import functools

import jax
import jax.numpy as jnp
import numpy as np
from jax import lax
from jax.experimental import pallas as pl
from jax.experimental.pallas import tpu as pltpu
from jax.experimental.pallas import tpu_sc as plsc

H = W = 50
P = 7
D = 10
S = 2
SCALE = 0.0625
DPAD = 16
R = 1000
NBINS = P * P
NG = NBINS * 16

NC, NS = 2, 16
NW = NC * NS
R_PAD = 1024
ROIS_PER_W = R_PAD // NW
STEP_ROIS = 2
NSTEP = ROIS_PER_W // STEP_ROIS
GPS = STEP_ROIS * NG
GCHUNK = 112
NCHUNK = NG // GCHUNK
OROWS = 56


def _repack_body(feat_ref, tab_ref):
    x = feat_ref[0, :, 0]
    xt = jnp.transpose(x, (1, 2, 0))
    tab_ref[0, 0] = jnp.concatenate(
        [xt, jnp.zeros((H, W, DPAD - D), jnp.float32)], axis=2)


def _repack(features):
    feats = features.reshape(2, D, NBINS, H, W)
    tab = pl.pallas_call(
        _repack_body,
        grid=(2, NBINS),
        in_specs=[pl.BlockSpec((1, D, 1, H, W), lambda b, g: (b, 0, g, 0, 0))],
        out_specs=pl.BlockSpec((1, 1, H, W, DPAD),
                               lambda b, g: (b, g, 0, 0, 0)),
        out_shape=jax.ShapeDtypeStruct((2, NBINS, H, W, DPAD), jnp.float32),
    )(feats)
    return tab.reshape(2 * NBINS * H * W, DPAD)


def _idxw_body(rois_ref, yoff_ref, xoff_ref, cy_ref, cx_ref, gb_ref,
               idx_ref, w_ref):
    b = rois_ref[:, 0:1].astype(jnp.int32)
    x1 = rois_ref[:, 1:2] * SCALE
    y1 = rois_ref[:, 2:3] * SCALE
    x2 = rois_ref[:, 3:4] * SCALE
    y2 = rois_ref[:, 4:5] * SCALE
    bin_w = jnp.maximum(x2 - x1, 0.1) / P
    bin_h = jnp.maximum(y2 - y1, 0.1) / P

    yoff = yoff_ref[:, :]
    xoff = xoff_ref[:, :]
    cy = cy_ref[:, :]
    cx = cx_ref[:, :]
    gb = gb_ref[:, :]

    y = jnp.clip(y1 + yoff * bin_h, 0.0, H - 1.0)
    x = jnp.clip(x1 + xoff * bin_w, 0.0, W - 1.0)
    y0f = jnp.floor(y)
    x0f = jnp.floor(x)
    wy = y - y0f
    wx = x - x0f
    y0 = y0f.astype(jnp.int32)
    x0 = x0f.astype(jnp.int32)
    yc = jnp.minimum(y0 + cy, H - 1)
    xc = jnp.minimum(x0 + cx, W - 1)
    wyf = jnp.where(cy > 0, wy, 1.0 - wy)
    wxf = jnp.where(cx > 0, wx, 1.0 - wx)

    idx_ref[:, :] = b * (NBINS * H * W) + gb + yc * W + xc
    w_ref[:, :] = 0.25 * wyf * wxf


def _make_tables():
    ph, pw, sy, sx, cy, cx = np.meshgrid(
        np.arange(P), np.arange(P), np.arange(S), np.arange(S),
        np.arange(2), np.arange(2), indexing="ij")
    yoff = ph + (sy + 0.5) / S
    xoff = pw + (sx + 0.5) / S
    gb = (ph * P + pw) * (H * W)
    flat = lambda a, dt: jnp.asarray(a.reshape(1, -1), dt)
    return (flat(yoff, jnp.float32), flat(xoff, jnp.float32),
            flat(cy, jnp.int32), flat(cx, jnp.int32), flat(gb, jnp.int32))


def _compute_idx_w(rois_p):
    tables = _make_tables()
    return pl.pallas_call(
        _idxw_body,
        out_shape=(jax.ShapeDtypeStruct((R_PAD, NG), jnp.int32),
                   jax.ShapeDtypeStruct((R_PAD, NG), jnp.float32)),
    )(rois_p, *tables)


def _sc_gather_accumulate(table, idx_hbm, w_hbm):
    mesh = plsc.VectorSubcoreMesh(core_axis_name="c", subcore_axis_name="s")
    cp = pltpu.CompilerParams()
    for f, v in (("needs_layout_passes", False),
                 ("use_tc_tiling_on_sc", False)):
        if f in pltpu.CompilerParams.__dataclass_fields__:
            cp = dataclasses.replace(cp, **{f: v})

    @functools.partial(
        pl.kernel,
        mesh=mesh,
        compiler_params=cp,
        out_type=jax.ShapeDtypeStruct((R_PAD * OROWS, DPAD), jnp.float32),
        scratch_types=[
            pltpu.VMEM((ROIS_PER_W, NG), jnp.int32),
            pltpu.VMEM((ROIS_PER_W, NG), jnp.float32),
            pltpu.VMEM((GPS, DPAD), jnp.float32),
            pltpu.VMEM((GPS, DPAD), jnp.float32),
            pltpu.VMEM((STEP_ROIS * OROWS, DPAD), jnp.float32),
            pltpu.SemaphoreType.DMA,
            pltpu.SemaphoreType.DMA,
        ],
    )
    def k(table_ref, idx_ref, w_ref, out_ref,
          idx_v, w_v, rows0, rows1, out_v, sem0, sem1):
        wid = lax.axis_index("s") * NC + lax.axis_index("c")
        rbase = pl.multiple_of(wid * ROIS_PER_W, ROIS_PER_W)

        pltpu.sync_copy(idx_ref.at[pl.ds(rbase, ROIS_PER_W)], idx_v)
        pltpu.sync_copy(w_ref.at[pl.ds(rbase, ROIS_PER_W)], w_v)

        def gathers(s, rows_v, sem):
            out = []
            for rr in range(STEP_ROIS):
                for kk in range(NCHUNK):
                    out.append(pltpu.make_async_copy(
                        table_ref.at[idx_v.at[s * STEP_ROIS + rr,
                                              pl.ds(kk * GCHUNK, GCHUNK)]],
                        rows_v.at[pl.ds(rr * NG + kk * GCHUNK, GCHUNK)],
                        sem))
            return out

        def compute_and_emit(s, rows_v):
            for rr in range(STEP_ROIS):
                lr = s * STEP_ROIS + rr
                rowvec = jnp.full((16,), lr, jnp.int32)

                @pl.loop(0, NBINS)
                def _bin(g):
                    acc = jnp.zeros((DPAD,), jnp.float32)
                    for j in range(16):
                        colvec = jnp.full((16,), g * 16 + j, jnp.int32)
                        wv = plsc.load_gather(w_v, [rowvec, colvec])
                        acc = acc + wv * rows_v[rr * NG + g * 16 + j, :]
                    out_v[rr * OROWS + g, :] = acc

            orow = pl.multiple_of((rbase + s * STEP_ROIS) * OROWS,
                                  STEP_ROIS * OROWS)
            pltpu.sync_copy(out_v, out_ref.at[pl.ds(orow, STEP_ROIS * OROWS)])

        for c in gathers(0, rows0, sem0):
            c.start()
        for c in gathers(0, rows0, sem0):
            c.wait()

        @pl.loop(0, NSTEP, step=2)
        def _steps(s):
            for c in gathers(s + 1, rows1, sem1):
                c.start()
            compute_and_emit(s, rows0)
            for c in gathers(s + 1, rows1, sem1):
                c.wait()
            @pl.when(s + 2 < NSTEP)
            def _issue():
                for c in gathers(s + 2, rows0, sem0):
                    c.start()
            compute_and_emit(s + 1, rows1)
            @pl.when(s + 2 < NSTEP)
            def _drain():
                for c in gathers(s + 2, rows0, sem0):
                    c.wait()

    return k(table, idx_hbm, w_hbm)


def _epilogue_body(rows_ref, out_ref):
    x = rows_ref[:, :, :]
    t = jnp.transpose(x, (0, 2, 1))
    out_ref[:, :, :] = t[:, :D, :NBINS]


def _epilogue(out_rows):
    rows = out_rows.reshape(R_PAD, OROWS, DPAD)
    out = pl.pallas_call(
        _epilogue_body,
        grid=(R // 8,),
        in_specs=[pl.BlockSpec((8, OROWS, DPAD), lambda r: (r, 0, 0))],
        out_specs=pl.BlockSpec((8, D, NBINS), lambda r: (r, 0, 0)),
        out_shape=jax.ShapeDtypeStruct((R, D, NBINS), jnp.float32),
    )(rows)
    return out.reshape(R, D, P, P)


def kernel(features, rois):
    table = _repack(features)

    rois_p = jnp.pad(rois, ((0, R_PAD - R), (0, 0)))
    idx, w = _compute_idx_w(rois_p)

    out_rows = _sc_gather_accumulate(table, idx, w)

    return _epilogue(out_rows)


# speedup vs baseline: 1.1427x; 1.1427x over previous
import dataclasses
import functools

import jax
import jax.numpy as jnp
import numpy as np
from jax import lax
from jax.experimental import pallas as pl
from jax.experimental.pallas import tpu as pltpu
from jax.experimental.pallas import tpu_sc as plsc

H = W = 50
P = 7
D = 10
S = 2
SCALE = 0.0625
DPAD = 16
R = 1000
NBINS = P * P
NG = NBINS * 16

NC, NS = 2, 16
NW = NC * NS
R_PAD = 1024
ROIS_PER_W = R_PAD // NW
STEP_ROIS = 2
NSTEP = ROIS_PER_W // STEP_ROIS
GPS = STEP_ROIS * NG
GCHUNK = 112
NCHUNK = NG // GCHUNK
OROWS = 56


M_ROWS = 2 * NBINS * H * W
PLANE = NBINS * H * W


GBLK = 7


def _repack_body(feat_ref, tab_ref):
    x = feat_ref[0]
    xt = jnp.transpose(x, (1, 2, 3, 0))
    tab_ref[0] = jnp.concatenate(
        [xt, jnp.zeros((GBLK, H, W, DPAD - D), jnp.float32)], axis=3)


def _repack(features):
    feats = features.reshape(2, D, NBINS, H, W)
    tab = pl.pallas_call(
        _repack_body,
        grid=(2, NBINS // GBLK),
        in_specs=[pl.BlockSpec((1, D, GBLK, H, W),
                               lambda b, g: (b, 0, g, 0, 0))],
        out_specs=pl.BlockSpec((1, GBLK, H, W, DPAD),
                               lambda b, g: (b, g, 0, 0, 0)),
        out_shape=jax.ShapeDtypeStruct((2, NBINS, H, W, DPAD), jnp.float32),
    )(feats)
    return tab.reshape(M_ROWS, DPAD)


def _idxw_body(rois_ref, yoff_ref, xoff_ref, cy_ref, cx_ref, gb_ref,
               idx_ref, w_ref):
    b = rois_ref[:, 0:1].astype(jnp.int32)
    x1 = rois_ref[:, 1:2] * SCALE
    y1 = rois_ref[:, 2:3] * SCALE
    x2 = rois_ref[:, 3:4] * SCALE
    y2 = rois_ref[:, 4:5] * SCALE
    bin_w = jnp.maximum(x2 - x1, 0.1) / P
    bin_h = jnp.maximum(y2 - y1, 0.1) / P

    yoff = yoff_ref[:, :]
    xoff = xoff_ref[:, :]
    cy = cy_ref[:, :]
    cx = cx_ref[:, :]
    gb = gb_ref[:, :]

    y = jnp.clip(y1 + yoff * bin_h, 0.0, H - 1.0)
    x = jnp.clip(x1 + xoff * bin_w, 0.0, W - 1.0)
    y0f = jnp.floor(y)
    x0f = jnp.floor(x)
    wy = y - y0f
    wx = x - x0f
    y0 = y0f.astype(jnp.int32)
    x0 = x0f.astype(jnp.int32)
    yc = jnp.minimum(y0 + cy, H - 1)
    xc = jnp.minimum(x0 + cx, W - 1)
    wyf = jnp.where(cy > 0, wy, 1.0 - wy)
    wxf = jnp.where(cx > 0, wx, 1.0 - wx)

    idx_ref[:, :] = b * (NBINS * H * W) + gb + yc * W + xc
    w_ref[:, :] = 0.25 * wyf * wxf


def _make_tables():
    ph, pw, sy, sx, cy, cx = np.meshgrid(
        np.arange(P), np.arange(P), np.arange(S), np.arange(S),
        np.arange(2), np.arange(2), indexing="ij")
    yoff = ph + (sy + 0.5) / S
    xoff = pw + (sx + 0.5) / S
    gb = (ph * P + pw) * (H * W)
    flat = lambda a, dt: jnp.asarray(a.reshape(1, -1), dt)
    return (flat(yoff, jnp.float32), flat(xoff, jnp.float32),
            flat(cy, jnp.int32), flat(cx, jnp.int32), flat(gb, jnp.int32))


def _compute_idx_w(rois_p):
    tables = _make_tables()
    return pl.pallas_call(
        _idxw_body,
        out_shape=(jax.ShapeDtypeStruct((R_PAD, NG), jnp.int32),
                   jax.ShapeDtypeStruct((R_PAD, NG), jnp.float32)),
    )(rois_p, *tables)


def _sc_gather_accumulate(table, idx_hbm, w_hbm):
    mesh = plsc.VectorSubcoreMesh(core_axis_name="c", subcore_axis_name="s")
    cp = pltpu.CompilerParams()
    for f, v in (("needs_layout_passes", False),
                 ("use_tc_tiling_on_sc", False)):
        if f in pltpu.CompilerParams.__dataclass_fields__:
            cp = dataclasses.replace(cp, **{f: v})

    @functools.partial(
        pl.kernel,
        mesh=mesh,
        compiler_params=cp,
        out_type=jax.ShapeDtypeStruct((R_PAD * OROWS, DPAD), jnp.float32),
        scratch_types=[
            pltpu.VMEM((ROIS_PER_W, NG), jnp.int32),
            pltpu.VMEM((ROIS_PER_W, NG), jnp.float32),
            pltpu.VMEM((GPS, DPAD), jnp.float32),
            pltpu.VMEM((GPS, DPAD), jnp.float32),
            pltpu.VMEM((STEP_ROIS * OROWS, DPAD), jnp.float32),
            pltpu.SemaphoreType.DMA,
            pltpu.SemaphoreType.DMA,
        ],
    )
    def k(table_ref, idx_ref, w_ref, out_ref,
          idx_v, w_v, rows0, rows1, out_v, sem0, sem1):
        wid = lax.axis_index("s") * NC + lax.axis_index("c")
        rbase = pl.multiple_of(wid * ROIS_PER_W, ROIS_PER_W)

        pltpu.sync_copy(idx_ref.at[pl.ds(rbase, ROIS_PER_W)], idx_v)
        pltpu.sync_copy(w_ref.at[pl.ds(rbase, ROIS_PER_W)], w_v)

        def gathers(s, rows_v, sem):
            out = []
            for rr in range(STEP_ROIS):
                for kk in range(NCHUNK):
                    out.append(pltpu.make_async_copy(
                        table_ref.at[idx_v.at[s * STEP_ROIS + rr,
                                              pl.ds(kk * GCHUNK, GCHUNK)]],
                        rows_v.at[pl.ds(rr * NG + kk * GCHUNK, GCHUNK)],
                        sem))
            return out

        def compute_and_emit(s, rows_v):
            for rr in range(STEP_ROIS):
                lr = s * STEP_ROIS + rr
                rowvec = jnp.full((16,), lr, jnp.int32)

                @pl.loop(0, NBINS)
                def _bin(g):
                    acc = jnp.zeros((DPAD,), jnp.float32)
                    for j in range(16):
                        colvec = jnp.full((16,), g * 16 + j, jnp.int32)
                        wv = plsc.load_gather(w_v, [rowvec, colvec])
                        acc = acc + wv * rows_v[rr * NG + g * 16 + j, :]
                    out_v[rr * OROWS + g, :] = acc

            orow = pl.multiple_of((rbase + s * STEP_ROIS) * OROWS,
                                  STEP_ROIS * OROWS)
            pltpu.sync_copy(out_v, out_ref.at[pl.ds(orow, STEP_ROIS * OROWS)])

        for c in gathers(0, rows0, sem0):
            c.start()
        for c in gathers(0, rows0, sem0):
            c.wait()

        @pl.loop(0, NSTEP, step=2)
        def _steps(s):
            for c in gathers(s + 1, rows1, sem1):
                c.start()
            compute_and_emit(s, rows0)
            for c in gathers(s + 1, rows1, sem1):
                c.wait()
            @pl.when(s + 2 < NSTEP)
            def _issue():
                for c in gathers(s + 2, rows0, sem0):
                    c.start()
            compute_and_emit(s + 1, rows1)
            @pl.when(s + 2 < NSTEP)
            def _drain():
                for c in gathers(s + 2, rows0, sem0):
                    c.wait()

    return k(table, idx_hbm, w_hbm)


def _epilogue_body(rows_ref, out_ref):
    x = rows_ref[:, :, :]
    t = jnp.transpose(x, (0, 2, 1))
    out_ref[:, :, :] = t[:, :D, :NBINS]


def _epilogue(out_rows):
    rows = out_rows.reshape(R_PAD, OROWS, DPAD)
    out = pl.pallas_call(
        _epilogue_body,
        grid=(R // 8,),
        in_specs=[pl.BlockSpec((8, OROWS, DPAD), lambda r: (r, 0, 0))],
        out_specs=pl.BlockSpec((8, D, NBINS), lambda r: (r, 0, 0)),
        out_shape=jax.ShapeDtypeStruct((R, D, NBINS), jnp.float32),
    )(rows)
    return out.reshape(R, D, P, P)


def kernel(features, rois):
    table = _repack(features)

    rois_p = jnp.pad(rois, ((0, R_PAD - R), (0, 0)))
    idx, w = _compute_idx_w(rois_p)

    out_rows = _sc_gather_accumulate(table, idx, w)

    return _epilogue(out_rows)


# speedup vs baseline: 1.2116x; 1.0603x over previous
import dataclasses
import functools

import jax
import jax.numpy as jnp
import numpy as np
from jax import lax
from jax.experimental import pallas as pl
from jax.experimental.pallas import tpu as pltpu
from jax.experimental.pallas import tpu_sc as plsc

H = W = 50
P = 7
D = 10
S = 2
SCALE = 0.0625
DPAD = 16
R = 1000
NBINS = P * P
NG = NBINS * 16

NC, NS = 2, 16
NW = NC * NS
R_PAD = 1024
ROIS_PER_W = R_PAD // NW
STEP_ROIS = 2
NSTEP = ROIS_PER_W // STEP_ROIS
GPS = STEP_ROIS * NG
GCHUNK = 112
NCHUNK = NG // GCHUNK
OROWS = 56


M_ROWS = 2 * NBINS * H * W
PLANE = NBINS * H * W


GROWS = 2 * H * W


def _repack_body(feat_ref, tab_ref):
    x = feat_ref[:, :, 0]
    xt = jnp.transpose(x, (0, 2, 3, 1))
    xr = xt.reshape(GROWS, D)
    tab_ref[:, :] = jnp.concatenate(
        [xr, jnp.zeros((GROWS, DPAD - D), jnp.float32)], axis=1)


def _repack(features):
    feats = features.reshape(2, D, NBINS, H, W)
    return pl.pallas_call(
        _repack_body,
        grid=(NBINS,),
        in_specs=[pl.BlockSpec((2, D, 1, H, W), lambda g: (0, 0, g, 0, 0))],
        out_specs=pl.BlockSpec((GROWS, DPAD), lambda g: (g, 0)),
        out_shape=jax.ShapeDtypeStruct((M_ROWS, DPAD), jnp.float32),
    )(feats)


def _idxw_body(rois_ref, yoff_ref, xoff_ref, cy_ref, cx_ref, gb_ref,
               idx_ref, w_ref):
    b = rois_ref[:, 0:1].astype(jnp.int32)
    x1 = rois_ref[:, 1:2] * SCALE
    y1 = rois_ref[:, 2:3] * SCALE
    x2 = rois_ref[:, 3:4] * SCALE
    y2 = rois_ref[:, 4:5] * SCALE
    bin_w = jnp.maximum(x2 - x1, 0.1) / P
    bin_h = jnp.maximum(y2 - y1, 0.1) / P

    yoff = yoff_ref[:, :]
    xoff = xoff_ref[:, :]
    cy = cy_ref[:, :]
    cx = cx_ref[:, :]
    gb = gb_ref[:, :]

    y = jnp.clip(y1 + yoff * bin_h, 0.0, H - 1.0)
    x = jnp.clip(x1 + xoff * bin_w, 0.0, W - 1.0)
    y0f = jnp.floor(y)
    x0f = jnp.floor(x)
    wy = y - y0f
    wx = x - x0f
    y0 = y0f.astype(jnp.int32)
    x0 = x0f.astype(jnp.int32)
    yc = jnp.minimum(y0 + cy, H - 1)
    xc = jnp.minimum(x0 + cx, W - 1)
    wyf = jnp.where(cy > 0, wy, 1.0 - wy)
    wxf = jnp.where(cx > 0, wx, 1.0 - wx)

    idx_ref[:, :] = gb + b * (H * W) + yc * W + xc
    w_ref[:, :] = 0.25 * wyf * wxf


def _make_tables():
    ph, pw, sy, sx, cy, cx = np.meshgrid(
        np.arange(P), np.arange(P), np.arange(S), np.arange(S),
        np.arange(2), np.arange(2), indexing="ij")
    yoff = ph + (sy + 0.5) / S
    xoff = pw + (sx + 0.5) / S
    gb = (ph * P + pw) * GROWS
    flat = lambda a, dt: jnp.asarray(a.reshape(1, -1), dt)
    return (flat(yoff, jnp.float32), flat(xoff, jnp.float32),
            flat(cy, jnp.int32), flat(cx, jnp.int32), flat(gb, jnp.int32))


def _compute_idx_w(rois_p):
    tables = _make_tables()
    return pl.pallas_call(
        _idxw_body,
        out_shape=(jax.ShapeDtypeStruct((R_PAD, NG), jnp.int32),
                   jax.ShapeDtypeStruct((R_PAD, NG), jnp.float32)),
    )(rois_p, *tables)


def _sc_gather_accumulate(table, idx_hbm, w_hbm):
    mesh = plsc.VectorSubcoreMesh(core_axis_name="c", subcore_axis_name="s")
    cp = pltpu.CompilerParams()
    for f, v in (("needs_layout_passes", False),
                 ("use_tc_tiling_on_sc", False)):
        if f in pltpu.CompilerParams.__dataclass_fields__:
            cp = dataclasses.replace(cp, **{f: v})

    @functools.partial(
        pl.kernel,
        mesh=mesh,
        compiler_params=cp,
        out_type=jax.ShapeDtypeStruct((R_PAD * OROWS, DPAD), jnp.float32),
        scratch_types=[
            pltpu.VMEM((ROIS_PER_W, NG), jnp.int32),
            pltpu.VMEM((ROIS_PER_W, NG), jnp.float32),
            pltpu.VMEM((GPS, DPAD), jnp.float32),
            pltpu.VMEM((GPS, DPAD), jnp.float32),
            pltpu.VMEM((STEP_ROIS * OROWS, DPAD), jnp.float32),
            pltpu.SemaphoreType.DMA,
            pltpu.SemaphoreType.DMA,
        ],
    )
    def k(table_ref, idx_ref, w_ref, out_ref,
          idx_v, w_v, rows0, rows1, out_v, sem0, sem1):
        wid = lax.axis_index("s") * NC + lax.axis_index("c")
        rbase = pl.multiple_of(wid * ROIS_PER_W, ROIS_PER_W)

        pltpu.sync_copy(idx_ref.at[pl.ds(rbase, ROIS_PER_W)], idx_v)
        pltpu.sync_copy(w_ref.at[pl.ds(rbase, ROIS_PER_W)], w_v)

        def gathers(s, rows_v, sem):
            out = []
            for rr in range(STEP_ROIS):
                for kk in range(NCHUNK):
                    out.append(pltpu.make_async_copy(
                        table_ref.at[idx_v.at[s * STEP_ROIS + rr,
                                              pl.ds(kk * GCHUNK, GCHUNK)]],
                        rows_v.at[pl.ds(rr * NG + kk * GCHUNK, GCHUNK)],
                        sem))
            return out

        def compute_and_emit(s, rows_v):
            for rr in range(STEP_ROIS):
                lr = s * STEP_ROIS + rr
                rowvec = jnp.full((16,), lr, jnp.int32)

                @pl.loop(0, NBINS)
                def _bin(g):
                    acc = jnp.zeros((DPAD,), jnp.float32)
                    for j in range(16):
                        colvec = jnp.full((16,), g * 16 + j, jnp.int32)
                        wv = plsc.load_gather(w_v, [rowvec, colvec])
                        acc = acc + wv * rows_v[rr * NG + g * 16 + j, :]
                    out_v[rr * OROWS + g, :] = acc

            orow = pl.multiple_of((rbase + s * STEP_ROIS) * OROWS,
                                  STEP_ROIS * OROWS)
            pltpu.sync_copy(out_v, out_ref.at[pl.ds(orow, STEP_ROIS * OROWS)])

        for c in gathers(0, rows0, sem0):
            c.start()
        for c in gathers(0, rows0, sem0):
            c.wait()

        @pl.loop(0, NSTEP, step=2)
        def _steps(s):
            for c in gathers(s + 1, rows1, sem1):
                c.start()
            compute_and_emit(s, rows0)
            for c in gathers(s + 1, rows1, sem1):
                c.wait()
            @pl.when(s + 2 < NSTEP)
            def _issue():
                for c in gathers(s + 2, rows0, sem0):
                    c.start()
            compute_and_emit(s + 1, rows1)
            @pl.when(s + 2 < NSTEP)
            def _drain():
                for c in gathers(s + 2, rows0, sem0):
                    c.wait()

    return k(table, idx_hbm, w_hbm)


RB = 200


def _epilogue_body(rows_ref, out_ref):
    x = rows_ref[:, :].reshape(RB, OROWS, DPAD)
    t = jnp.transpose(x, (0, 2, 1))
    out_ref[:, :, :] = t[:, :D, :NBINS]


def _epilogue(out_rows):
    out = pl.pallas_call(
        _epilogue_body,
        grid=(R // RB,),
        in_specs=[pl.BlockSpec((RB * OROWS, DPAD), lambda r: (r, 0))],
        out_specs=pl.BlockSpec((RB, D, NBINS), lambda r: (r, 0, 0)),
        out_shape=jax.ShapeDtypeStruct((R, D, NBINS), jnp.float32),
    )(out_rows)
    return out.reshape(R, D, P, P)


def kernel(features, rois):
    table = _repack(features)

    rois_p = jnp.pad(rois, ((0, R_PAD - R), (0, 0)))
    idx, w = _compute_idx_w(rois_p)

    out_rows = _sc_gather_accumulate(table, idx, w)

    return _epilogue(out_rows)


# speedup vs baseline: 1.2145x; 1.0025x over previous
import dataclasses
import functools

import jax
import jax.numpy as jnp
import numpy as np
from jax import lax
from jax.experimental import pallas as pl
from jax.experimental.pallas import tpu as pltpu
from jax.experimental.pallas import tpu_sc as plsc

H = W = 50
P = 7
D = 10
S = 2
SCALE = 0.0625
DPAD = 16
R = 1000
NBINS = P * P
NG = NBINS * 16

NC, NS = 2, 16
NW = NC * NS
R_PAD = 1024
ROIS_PER_W = R_PAD // NW
STEP_ROIS = 2
NSTEP = ROIS_PER_W // STEP_ROIS
GPS = STEP_ROIS * NG
GCHUNK = 112
NCHUNK = NG // GCHUNK
OROWS = 56


M_ROWS = 2 * NBINS * H * W
PLANE = NBINS * H * W


GROWS = 2 * H * W


GB7 = 7


def _repack_body(feat_ref, tab_ref):
    x = feat_ref[:, :, :]
    xt = jnp.transpose(x, (2, 0, 3, 4, 1))
    xr = xt.reshape(GB7 * GROWS, D)
    tab_ref[:, :] = jnp.concatenate(
        [xr, jnp.zeros((GB7 * GROWS, DPAD - D), jnp.float32)], axis=1)


def _repack(features):
    feats = features.reshape(2, D, NBINS, H, W)
    return pl.pallas_call(
        _repack_body,
        grid=(NBINS // GB7,),
        in_specs=[pl.BlockSpec((2, D, GB7, H, W),
                               lambda g: (0, 0, g, 0, 0))],
        out_specs=pl.BlockSpec((GB7 * GROWS, DPAD), lambda g: (g, 0)),
        out_shape=jax.ShapeDtypeStruct((M_ROWS, DPAD), jnp.float32),
    )(feats)


def _idxw_body(rois_ref, yoff_ref, xoff_ref, cy_ref, cx_ref, gb_ref,
               idx_ref, w_ref):
    b = rois_ref[:, 0:1].astype(jnp.int32)
    x1 = rois_ref[:, 1:2] * SCALE
    y1 = rois_ref[:, 2:3] * SCALE
    x2 = rois_ref[:, 3:4] * SCALE
    y2 = rois_ref[:, 4:5] * SCALE
    bin_w = jnp.maximum(x2 - x1, 0.1) / P
    bin_h = jnp.maximum(y2 - y1, 0.1) / P

    yoff = yoff_ref[:, :]
    xoff = xoff_ref[:, :]
    cy = cy_ref[:, :]
    cx = cx_ref[:, :]
    gb = gb_ref[:, :]

    y = jnp.clip(y1 + yoff * bin_h, 0.0, H - 1.0)
    x = jnp.clip(x1 + xoff * bin_w, 0.0, W - 1.0)
    y0f = jnp.floor(y)
    x0f = jnp.floor(x)
    wy = y - y0f
    wx = x - x0f
    y0 = y0f.astype(jnp.int32)
    x0 = x0f.astype(jnp.int32)
    yc = jnp.minimum(y0 + cy, H - 1)
    xc = jnp.minimum(x0 + cx, W - 1)
    wyf = jnp.where(cy > 0, wy, 1.0 - wy)
    wxf = jnp.where(cx > 0, wx, 1.0 - wx)

    idx_ref[:, :] = gb + b * (H * W) + yc * W + xc
    w_ref[:, :] = 0.25 * wyf * wxf


def _make_tables():
    ph, pw, sy, sx, cy, cx = np.meshgrid(
        np.arange(P), np.arange(P), np.arange(S), np.arange(S),
        np.arange(2), np.arange(2), indexing="ij")
    yoff = ph + (sy + 0.5) / S
    xoff = pw + (sx + 0.5) / S
    gb = (ph * P + pw) * GROWS
    flat = lambda a, dt: jnp.asarray(a.reshape(1, -1), dt)
    return (flat(yoff, jnp.float32), flat(xoff, jnp.float32),
            flat(cy, jnp.int32), flat(cx, jnp.int32), flat(gb, jnp.int32))


def _compute_idx_w(rois_p):
    tables = _make_tables()
    return pl.pallas_call(
        _idxw_body,
        out_shape=(jax.ShapeDtypeStruct((R_PAD, NG), jnp.int32),
                   jax.ShapeDtypeStruct((R_PAD, NG), jnp.float32)),
    )(rois_p, *tables)


def _sc_gather_accumulate(table, idx_hbm, w_hbm):
    mesh = plsc.VectorSubcoreMesh(core_axis_name="c", subcore_axis_name="s")
    cp = pltpu.CompilerParams()
    for f, v in (("needs_layout_passes", False),
                 ("use_tc_tiling_on_sc", False)):
        if f in pltpu.CompilerParams.__dataclass_fields__:
            cp = dataclasses.replace(cp, **{f: v})

    @functools.partial(
        pl.kernel,
        mesh=mesh,
        compiler_params=cp,
        out_type=jax.ShapeDtypeStruct((R_PAD * OROWS, DPAD), jnp.float32),
        scratch_types=[
            pltpu.VMEM((ROIS_PER_W, NG), jnp.int32),
            pltpu.VMEM((ROIS_PER_W, NG), jnp.float32),
            pltpu.VMEM((GPS, DPAD), jnp.float32),
            pltpu.VMEM((GPS, DPAD), jnp.float32),
            pltpu.VMEM((STEP_ROIS * OROWS, DPAD), jnp.float32),
            pltpu.SemaphoreType.DMA,
            pltpu.SemaphoreType.DMA,
        ],
    )
    def k(table_ref, idx_ref, w_ref, out_ref,
          idx_v, w_v, rows0, rows1, out_v, sem0, sem1):
        wid = lax.axis_index("s") * NC + lax.axis_index("c")
        rbase = pl.multiple_of(wid * ROIS_PER_W, ROIS_PER_W)

        pltpu.sync_copy(idx_ref.at[pl.ds(rbase, ROIS_PER_W)], idx_v)
        pltpu.sync_copy(w_ref.at[pl.ds(rbase, ROIS_PER_W)], w_v)

        def gathers(s, rows_v, sem):
            out = []
            for rr in range(STEP_ROIS):
                for kk in range(NCHUNK):
                    out.append(pltpu.make_async_copy(
                        table_ref.at[idx_v.at[s * STEP_ROIS + rr,
                                              pl.ds(kk * GCHUNK, GCHUNK)]],
                        rows_v.at[pl.ds(rr * NG + kk * GCHUNK, GCHUNK)],
                        sem))
            return out

        def compute_and_emit(s, rows_v):
            for rr in range(STEP_ROIS):
                lr = s * STEP_ROIS + rr
                rowvec = jnp.full((16,), lr, jnp.int32)

                @pl.loop(0, NBINS)
                def _bin(g):
                    acc = jnp.zeros((DPAD,), jnp.float32)
                    for j in range(16):
                        colvec = jnp.full((16,), g * 16 + j, jnp.int32)
                        wv = plsc.load_gather(w_v, [rowvec, colvec])
                        acc = acc + wv * rows_v[rr * NG + g * 16 + j, :]
                    out_v[rr * OROWS + g, :] = acc

            orow = pl.multiple_of((rbase + s * STEP_ROIS) * OROWS,
                                  STEP_ROIS * OROWS)
            pltpu.sync_copy(out_v, out_ref.at[pl.ds(orow, STEP_ROIS * OROWS)])

        for c in gathers(0, rows0, sem0):
            c.start()
        for c in gathers(0, rows0, sem0):
            c.wait()

        @pl.loop(0, NSTEP, step=2)
        def _steps(s):
            for c in gathers(s + 1, rows1, sem1):
                c.start()
            compute_and_emit(s, rows0)
            for c in gathers(s + 1, rows1, sem1):
                c.wait()
            @pl.when(s + 2 < NSTEP)
            def _issue():
                for c in gathers(s + 2, rows0, sem0):
                    c.start()
            compute_and_emit(s + 1, rows1)
            @pl.when(s + 2 < NSTEP)
            def _drain():
                for c in gathers(s + 2, rows0, sem0):
                    c.wait()

    return k(table, idx_hbm, w_hbm)


RB = 200


def _epilogue_body(rows_ref, out_ref):
    x = rows_ref[:, :].reshape(RB, OROWS, DPAD)
    t = jnp.transpose(x, (0, 2, 1))
    out_ref[:, :, :] = t[:, :D, :NBINS]


def _epilogue(out_rows):
    out = pl.pallas_call(
        _epilogue_body,
        grid=(R // RB,),
        in_specs=[pl.BlockSpec((RB * OROWS, DPAD), lambda r: (r, 0))],
        out_specs=pl.BlockSpec((RB, D, NBINS), lambda r: (r, 0, 0)),
        out_shape=jax.ShapeDtypeStruct((R, D, NBINS), jnp.float32),
    )(out_rows)
    return out.reshape(R, D, P, P)


def kernel(features, rois):
    table = _repack(features)

    rois_p = jnp.pad(rois, ((0, R_PAD - R), (0, 0)))
    idx, w = _compute_idx_w(rois_p)

    out_rows = _sc_gather_accumulate(table, idx, w)

    return _epilogue(out_rows)
